# baseline (speedup 1.0000x reference)
import functools

import jax
import jax.numpy as jnp
from jax import lax
from jax.experimental import pallas as pl
from jax.experimental.pallas import tpu as pltpu
from jax.experimental.pallas import tpu_sc as plsc

N = 2097152
D = 64
BINS = 65
NC = 2
NS = 16
L = 16
NW = NC * NS
VPW = N // NW
CW = 8192
CH = VPW // CW
HW = BINS * L

RB = 2048
GRID = N // RB


def _tc_body(a_ref, b_ref, out_ref):
    out_ref[...] = jnp.sum(jnp.abs(a_ref[...] - b_ref[...]), axis=1)


_tc_distances = pl.pallas_call(
    _tc_body,
    grid=(GRID,),
    in_specs=[
        pl.BlockSpec((RB, D), lambda i: (i, 0)),
        pl.BlockSpec((RB, D), lambda i: (i, 0)),
    ],
    out_specs=pl.BlockSpec((RB,), lambda i: (i,)),
    out_shape=jax.ShapeDtypeStruct((N,), jnp.float32),
)

_mesh = plsc.VectorSubcoreMesh(
    core_axis_name="c", subcore_axis_name="s", num_cores=NC, num_subcores=NS
)


@functools.partial(
    pl.kernel,
    out_type=[
        jax.ShapeDtypeStruct((NW, HW), jnp.float32),
        jax.ShapeDtypeStruct((NW, L), jnp.float32),
    ],
    mesh=_mesh,
    compiler_params=pltpu.CompilerParams(needs_layout_passes=False),
    scratch_types=[
        pltpu.VMEM((CW,), jnp.float32),
        pltpu.VMEM((CW,), jnp.float32),
        pltpu.VMEM((HW,), jnp.float32),
        pltpu.VMEM((L,), jnp.float32),
        pltpu.SemaphoreType.DMA,
        pltpu.SemaphoreType.DMA,
    ],
)
def _sc_hist(d_hbm, hist_out, sum_out, d0, d1, hist_v, sum_v, s0, s1):
    wid = lax.axis_index("s") * NC + lax.axis_index("c")
    base = wid * VPW

    zeros = jnp.zeros((L,), jnp.float32)
    ones = jnp.ones((L,), jnp.float32)
    lane = lax.iota(jnp.int32, L)

    for i in range(BINS):
        hist_v[pl.ds(i * L, L)] = zeros
    sum_v[...] = zeros

    def issue(g, dref, sem):
        pltpu.async_copy(d_hbm.at[pl.ds(base + g * CW, CW)], dref, sem)

    def wait(dref, sem):
        pltpu.make_async_copy(d_hbm.at[pl.ds(0, CW)], dref, sem).wait()

    def compute(dref):
        hist_t = hist_v.at[pl.ds(0, HW)]
        sum_t = sum_v.at[pl.ds(0, L)]

        def step(i, carry):
            for u in range(8):
                dv = dref[pl.ds((i * 8 + u) * L, L)]
                di = dv.astype(jnp.int32)
                plsc.addupdate_scatter(hist_t, [di * L + lane], ones)
                plsc.addupdate(sum_t, dv)
            return carry
        lax.fori_loop(0, CW // (8 * L), step, 0)

    issue(0, d0, s0)
    issue(1, d1, s1)

    def outer(t, carry):
        g = t * 2
        wait(d0, s0)
        compute(d0)

        @pl.when(g + 2 < CH)
        def _():
            issue(g + 2, d0, s0)

        wait(d1, s1)
        compute(d1)

        @pl.when(g + 3 < CH)
        def _():
            issue(g + 3, d1, s1)

        return carry

    lax.fori_loop(0, CH // 2, outer, 0)

    pltpu.sync_copy(hist_v, hist_out.at[wid])
    pltpu.sync_copy(sum_v, sum_out.at[wid])


def kernel(y_pred, y_true):
    distances = _tc_distances(y_pred, y_true)
    hist_parts, sum_parts = _sc_hist(distances)
    histogram = hist_parts.reshape(NW, BINS, L).sum(axis=(0, 2))
    mean = sum_parts.sum() / jnp.float32(N)
    return mean, histogram


# speedup vs baseline: 3.6480x; 3.6480x over previous
import functools

import jax
import jax.numpy as jnp
from jax import lax
from jax.experimental import pallas as pl
from jax.experimental.pallas import tpu as pltpu
from jax.experimental.pallas import tpu_sc as plsc

N = 2097152
D = 64
BINS = 65
NC = 2
NS = 16
L = 16
NW = NC * NS
VPW = N // NW
CW = 8192
CH = VPW // CW
HW = BINS * L

CB = 2048
GRID = N // CB


def _tc_body(a_ref, b_ref, out_ref):
    out_ref[...] = jnp.sum(jnp.abs(a_ref[...] - b_ref[...]), axis=0)


_tc_distances = pl.pallas_call(
    _tc_body,
    grid=(GRID,),
    in_specs=[
        pl.BlockSpec((D, CB), lambda i: (0, i)),
        pl.BlockSpec((D, CB), lambda i: (0, i)),
    ],
    out_specs=pl.BlockSpec((CB,), lambda i: (i,)),
    out_shape=jax.ShapeDtypeStruct((N,), jnp.float32),
)

_mesh = plsc.VectorSubcoreMesh(
    core_axis_name="c", subcore_axis_name="s", num_cores=NC, num_subcores=NS
)


@functools.partial(
    pl.kernel,
    out_type=[
        jax.ShapeDtypeStruct((NW, HW), jnp.float32),
        jax.ShapeDtypeStruct((NW, L), jnp.float32),
    ],
    mesh=_mesh,
    compiler_params=pltpu.CompilerParams(needs_layout_passes=False),
    scratch_types=[
        pltpu.VMEM((CW,), jnp.float32),
        pltpu.VMEM((CW,), jnp.float32),
        pltpu.VMEM((HW,), jnp.float32),
        pltpu.VMEM((L,), jnp.float32),
        pltpu.SemaphoreType.DMA,
        pltpu.SemaphoreType.DMA,
    ],
)
def _sc_hist(d_hbm, hist_out, sum_out, d0, d1, hist_v, sum_v, s0, s1):
    wid = lax.axis_index("s") * NC + lax.axis_index("c")
    base = wid * VPW

    zeros = jnp.zeros((L,), jnp.float32)
    ones = jnp.ones((L,), jnp.float32)
    lane = lax.iota(jnp.int32, L)

    for i in range(BINS):
        hist_v[pl.ds(i * L, L)] = zeros
    sum_v[...] = zeros

    def issue(g, dref, sem):
        pltpu.async_copy(d_hbm.at[pl.ds(base + g * CW, CW)], dref, sem)

    def wait(dref, sem):
        pltpu.make_async_copy(d_hbm.at[pl.ds(0, CW)], dref, sem).wait()

    def compute(dref):
        hist_t = hist_v.at[pl.ds(0, HW)]
        sum_t = sum_v.at[pl.ds(0, L)]

        def step(i, carry):
            for u in range(8):
                dv = dref[pl.ds((i * 8 + u) * L, L)]
                di = dv.astype(jnp.int32)
                plsc.addupdate_scatter(hist_t, [di * L + lane], ones)
                plsc.addupdate(sum_t, dv)
            return carry
        lax.fori_loop(0, CW // (8 * L), step, 0)

    issue(0, d0, s0)
    issue(1, d1, s1)

    def outer(t, carry):
        g = t * 2
        wait(d0, s0)
        compute(d0)

        @pl.when(g + 2 < CH)
        def _():
            issue(g + 2, d0, s0)

        wait(d1, s1)
        compute(d1)

        @pl.when(g + 3 < CH)
        def _():
            issue(g + 3, d1, s1)

        return carry

    lax.fori_loop(0, CH // 2, outer, 0)

    pltpu.sync_copy(hist_v, hist_out.at[wid])
    pltpu.sync_copy(sum_v, sum_out.at[wid])


def kernel(y_pred, y_true):
    distances = _tc_distances(y_pred.T, y_true.T)
    hist_parts, sum_parts = _sc_hist(distances)
    histogram = hist_parts.reshape(NW, BINS, L).sum(axis=(0, 2))
    mean = sum_parts.sum() / jnp.float32(N)
    return mean, histogram


# speedup vs baseline: 6.9773x; 1.9126x over previous
import functools

import jax
import jax.numpy as jnp
from jax import lax
from jax.experimental import pallas as pl
from jax.experimental.pallas import tpu as pltpu
from jax.experimental.pallas import tpu_sc as plsc

N = 2097152
D = 64
BINS = 65
NC = 2
NS = 16
L = 16
NW = NC * NS
VPW = N // NW
CW = 8192
CH = VPW // CW
HW = BINS * L

CB = 8192
GRID = N // CB


def _tc_body(a_ref, b_ref, out_ref):
    ai = a_ref[...].view(jnp.int32)
    bi = b_ref[...].view(jnp.int32)
    out_ref[...] = jnp.sum((ai ^ bi).view(jnp.float32), axis=0)


_tc_distances = pl.pallas_call(
    _tc_body,
    grid=(GRID,),
    in_specs=[
        pl.BlockSpec((D, CB), lambda i: (0, i)),
        pl.BlockSpec((D, CB), lambda i: (0, i)),
    ],
    out_specs=pl.BlockSpec((CB,), lambda i: (i,)),
    out_shape=jax.ShapeDtypeStruct((N,), jnp.float32),
)

_mesh = plsc.VectorSubcoreMesh(
    core_axis_name="c", subcore_axis_name="s", num_cores=NC, num_subcores=NS
)


@functools.partial(
    pl.kernel,
    out_type=[
        jax.ShapeDtypeStruct((NW, HW), jnp.float32),
        jax.ShapeDtypeStruct((NW, L), jnp.float32),
    ],
    mesh=_mesh,
    compiler_params=pltpu.CompilerParams(needs_layout_passes=False),
    scratch_types=[
        pltpu.VMEM((CW,), jnp.float32),
        pltpu.VMEM((CW,), jnp.float32),
        pltpu.VMEM((HW,), jnp.float32),
        pltpu.VMEM((L,), jnp.float32),
        pltpu.SemaphoreType.DMA,
        pltpu.SemaphoreType.DMA,
    ],
)
def _sc_hist(d_hbm, hist_out, sum_out, d0, d1, hist_v, sum_v, s0, s1):
    wid = lax.axis_index("s") * NC + lax.axis_index("c")
    base = wid * VPW

    zeros = jnp.zeros((L,), jnp.float32)
    ones = jnp.ones((L,), jnp.float32)
    lane = lax.iota(jnp.int32, L)

    for i in range(BINS):
        hist_v[pl.ds(i * L, L)] = zeros
    sum_v[...] = zeros

    def issue(g, dref, sem):
        pltpu.async_copy(d_hbm.at[pl.ds(base + g * CW, CW)], dref, sem)

    def wait(dref, sem):
        pltpu.make_async_copy(d_hbm.at[pl.ds(0, CW)], dref, sem).wait()

    def compute(dref):
        hist_t = hist_v.at[pl.ds(0, HW)]
        sum_t = sum_v.at[pl.ds(0, L)]

        def step(i, carry):
            for u in range(8):
                dv = dref[pl.ds((i * 8 + u) * L, L)]
                di = dv.astype(jnp.int32)
                plsc.addupdate_scatter(hist_t, [di * L + lane], ones)
                plsc.addupdate(sum_t, dv)
            return carry
        lax.fori_loop(0, CW // (8 * L), step, 0)

    issue(0, d0, s0)
    issue(1, d1, s1)

    def outer(t, carry):
        g = t * 2
        wait(d0, s0)
        compute(d0)

        @pl.when(g + 2 < CH)
        def _():
            issue(g + 2, d0, s0)

        wait(d1, s1)
        compute(d1)

        @pl.when(g + 3 < CH)
        def _():
            issue(g + 3, d1, s1)

        return carry

    lax.fori_loop(0, CH // 2, outer, 0)

    pltpu.sync_copy(hist_v, hist_out.at[wid])
    pltpu.sync_copy(sum_v, sum_out.at[wid])


def kernel(y_pred, y_true):
    distances = _tc_distances(y_pred.T, y_true.T)
    hist_parts, sum_parts = _sc_hist(distances)
    histogram = hist_parts.reshape(NW, BINS, L).sum(axis=(0, 2))
    mean = sum_parts.sum() / jnp.float32(N)
    return mean, histogram


# speedup vs baseline: 7.3404x; 1.0520x over previous
import functools

import jax
import jax.numpy as jnp
from jax import lax
from jax.experimental import pallas as pl
from jax.experimental.pallas import tpu as pltpu
from jax.experimental.pallas import tpu_sc as plsc

N = 2097152
D = 64
BINS = 65
NC = 2
NS = 16
L = 16
NW = NC * NS
VPW = N // NW
CW = 8192
CH = VPW // CW
HW = BINS * L

CB = 16384
GRID = N // CB


def _tc_body(a_ref, b_ref, out_ref):
    ai = a_ref[...].view(jnp.int32)
    bi = b_ref[...].view(jnp.int32)
    out_ref[...] = jnp.sum((ai ^ bi).view(jnp.float32), axis=0)


_tc_distances = pl.pallas_call(
    _tc_body,
    grid=(GRID,),
    in_specs=[
        pl.BlockSpec((D, CB), lambda i: (0, i)),
        pl.BlockSpec((D, CB), lambda i: (0, i)),
    ],
    out_specs=pl.BlockSpec((CB,), lambda i: (i,)),
    out_shape=jax.ShapeDtypeStruct((N,), jnp.float32),
)

_mesh = plsc.VectorSubcoreMesh(
    core_axis_name="c", subcore_axis_name="s", num_cores=NC, num_subcores=NS
)


@functools.partial(
    pl.kernel,
    out_type=[
        jax.ShapeDtypeStruct((NW, HW), jnp.float32),
        jax.ShapeDtypeStruct((NW, L), jnp.float32),
    ],
    mesh=_mesh,
    compiler_params=pltpu.CompilerParams(needs_layout_passes=False),
    scratch_types=[
        pltpu.VMEM((CW,), jnp.float32),
        pltpu.VMEM((CW,), jnp.float32),
        pltpu.VMEM((HW,), jnp.float32),
        pltpu.VMEM((L,), jnp.float32),
        pltpu.SemaphoreType.DMA,
        pltpu.SemaphoreType.DMA,
    ],
)
def _sc_hist(d_hbm, hist_out, sum_out, d0, d1, hist_v, sum_v, s0, s1):
    wid = lax.axis_index("s") * NC + lax.axis_index("c")
    base = wid * VPW

    zeros = jnp.zeros((L,), jnp.float32)
    ones = jnp.ones((L,), jnp.float32)
    lane = lax.iota(jnp.int32, L)

    for i in range(BINS):
        hist_v[pl.ds(i * L, L)] = zeros
    sum_v[...] = zeros

    def issue(g, dref, sem):
        pltpu.async_copy(d_hbm.at[pl.ds(base + g * CW, CW)], dref, sem)

    def wait(dref, sem):
        pltpu.make_async_copy(d_hbm.at[pl.ds(0, CW)], dref, sem).wait()

    def compute(dref):
        hist_t = hist_v.at[pl.ds(0, HW)]
        sum_t = sum_v.at[pl.ds(0, L)]

        def step(i, carry):
            for u in range(8):
                dv = dref[pl.ds((i * 8 + u) * L, L)]
                di = dv.astype(jnp.int32)
                plsc.addupdate_scatter(hist_t, [di * L + lane], ones)
                plsc.addupdate(sum_t, dv)
            return carry
        lax.fori_loop(0, CW // (8 * L), step, 0)

    issue(0, d0, s0)
    issue(1, d1, s1)

    def outer(t, carry):
        g = t * 2
        wait(d0, s0)
        compute(d0)

        @pl.when(g + 2 < CH)
        def _():
            issue(g + 2, d0, s0)

        wait(d1, s1)
        compute(d1)

        @pl.when(g + 3 < CH)
        def _():
            issue(g + 3, d1, s1)

        return carry

    lax.fori_loop(0, CH // 2, outer, 0)

    pltpu.sync_copy(hist_v, hist_out.at[wid])
    pltpu.sync_copy(sum_v, sum_out.at[wid])


def kernel(y_pred, y_true):
    distances = _tc_distances(y_pred.T, y_true.T)
    hist_parts, sum_parts = _sc_hist(distances)
    histogram = hist_parts.reshape(NW, BINS, L).sum(axis=(0, 2))
    mean = sum_parts.sum() / jnp.float32(N)
    return mean, histogram


# speedup vs baseline: 7.5230x; 1.0249x over previous
import functools

import jax
import jax.numpy as jnp
from jax import lax
from jax.experimental import pallas as pl
from jax.experimental.pallas import tpu as pltpu
from jax.experimental.pallas import tpu_sc as plsc

N = 2097152
D = 64
BINS = 65
NC = 2
NS = 16
L = 16
NW = NC * NS
VPW = N // NW
CW = 8192
CH = VPW // CW
HW = BINS * L

CB = 32768
GRID = N // CB


def _tc_body(a_ref, b_ref, out_ref):
    ai = a_ref[...].view(jnp.int32)
    bi = b_ref[...].view(jnp.int32)
    out_ref[...] = jnp.sum((ai ^ bi).view(jnp.float32), axis=0)


_tc_distances = pl.pallas_call(
    _tc_body,
    grid=(GRID,),
    in_specs=[
        pl.BlockSpec((D, CB), lambda i: (0, i)),
        pl.BlockSpec((D, CB), lambda i: (0, i)),
    ],
    out_specs=pl.BlockSpec((CB,), lambda i: (i,)),
    out_shape=jax.ShapeDtypeStruct((N,), jnp.float32),
)

_mesh = plsc.VectorSubcoreMesh(
    core_axis_name="c", subcore_axis_name="s", num_cores=NC, num_subcores=NS
)


@functools.partial(
    pl.kernel,
    out_type=[
        jax.ShapeDtypeStruct((NW, HW), jnp.float32),
        jax.ShapeDtypeStruct((NW, L), jnp.float32),
    ],
    mesh=_mesh,
    compiler_params=pltpu.CompilerParams(needs_layout_passes=False),
    scratch_types=[
        pltpu.VMEM((CW,), jnp.float32),
        pltpu.VMEM((CW,), jnp.float32),
        pltpu.VMEM((HW,), jnp.float32),
        pltpu.VMEM((L,), jnp.float32),
        pltpu.SemaphoreType.DMA,
        pltpu.SemaphoreType.DMA,
    ],
)
def _sc_hist(d_hbm, hist_out, sum_out, d0, d1, hist_v, sum_v, s0, s1):
    wid = lax.axis_index("s") * NC + lax.axis_index("c")
    base = wid * VPW

    zeros = jnp.zeros((L,), jnp.float32)
    ones = jnp.ones((L,), jnp.float32)
    lane = lax.iota(jnp.int32, L)

    for i in range(BINS):
        hist_v[pl.ds(i * L, L)] = zeros
    sum_v[...] = zeros

    def issue(g, dref, sem):
        pltpu.async_copy(d_hbm.at[pl.ds(base + g * CW, CW)], dref, sem)

    def wait(dref, sem):
        pltpu.make_async_copy(d_hbm.at[pl.ds(0, CW)], dref, sem).wait()

    def compute(dref):
        hist_t = hist_v.at[pl.ds(0, HW)]
        sum_t = sum_v.at[pl.ds(0, L)]

        def step(i, carry):
            for u in range(8):
                dv = dref[pl.ds((i * 8 + u) * L, L)]
                di = dv.astype(jnp.int32)
                plsc.addupdate_scatter(hist_t, [di * L + lane], ones)
                plsc.addupdate(sum_t, dv)
            return carry
        lax.fori_loop(0, CW // (8 * L), step, 0)

    issue(0, d0, s0)
    issue(1, d1, s1)

    def outer(t, carry):
        g = t * 2
        wait(d0, s0)
        compute(d0)

        @pl.when(g + 2 < CH)
        def _():
            issue(g + 2, d0, s0)

        wait(d1, s1)
        compute(d1)

        @pl.when(g + 3 < CH)
        def _():
            issue(g + 3, d1, s1)

        return carry

    lax.fori_loop(0, CH // 2, outer, 0)

    pltpu.sync_copy(hist_v, hist_out.at[wid])
    pltpu.sync_copy(sum_v, sum_out.at[wid])


def kernel(y_pred, y_true):
    distances = _tc_distances(y_pred.T, y_true.T)
    hist_parts, sum_parts = _sc_hist(distances)
    histogram = hist_parts.reshape(NW, BINS, L).sum(axis=(0, 2))
    mean = sum_parts.sum() / jnp.float32(N)
    return mean, histogram
